# baseline (speedup 1.0000x reference)
import jax
import jax.numpy as jnp
from jax import lax
from jax.experimental import pallas as pl
from jax.experimental.pallas import tpu as pltpu
from jax.experimental.pallas import tpu_sc as plsc

MAX_LEN = 8192
DIM = 1024
NUM_WORKERS = 32
ROWS_PER_WORKER = MAX_LEN // NUM_WORKERS
CHUNK = 16
CHUNKS_PER_WORKER = ROWS_PER_WORKER // CHUNK
NBUF = 4

_mesh = plsc.VectorSubcoreMesh(core_axis_name="c", subcore_axis_name="s")


def _sc_body(t_hbm, pe_hbm, zeros_hbm, out_hbm,
             t_v, buf0, buf1, buf2, buf3,
             si0, si1, si2, si3, so0, so1, so2, so3):
    wid = lax.axis_index("s") * 2 + lax.axis_index("c")
    base = wid * ROWS_PER_WORKER

    pltpu.sync_copy(t_hbm, t_v)
    t = t_v[...][0]

    bufs = (buf0, buf1, buf2, buf3)
    sin = (si0, si1, si2, si3)
    sout = (so0, so1, so2, so3)

    @pl.when(base + ROWS_PER_WORKER <= t)
    def _fast_copy():
        n = CHUNKS_PER_WORKER
        h_in = [None] * n
        h_out = [None] * n
        for i in range(n):
            b = i % NBUF
            if i >= NBUF:
                h_out[i - NBUF].wait()
            src = pe_hbm.at[pl.ds(base + i * CHUNK, CHUNK)]
            h_in[i] = pltpu.async_copy(src, bufs[b], sin[b])
            if i >= 1:
                h_in[i - 1].wait()
                pb = (i - 1) % NBUF
                dst = out_hbm.at[pl.ds(base + (i - 1) * CHUNK, CHUNK)]
                h_out[i - 1] = pltpu.async_copy(bufs[pb], dst, sout[pb])
        h_in[n - 1].wait()
        lb = (n - 1) % NBUF
        dst = out_hbm.at[pl.ds(base + (n - 1) * CHUNK, CHUNK)]
        h_out[n - 1] = pltpu.async_copy(bufs[lb], dst, sout[lb])
        for i in range(n - NBUF, n):
            h_out[i].wait()

    @pl.when(base + ROWS_PER_WORKER > t)
    def _masked_path():
        for ci in range(CHUNKS_PER_WORKER):
            cbase = base + ci * CHUNK

            @pl.when(cbase + CHUNK <= t)
            def _copy_chunk():
                pltpu.sync_copy(pe_hbm.at[pl.ds(cbase, CHUNK)], buf0)
                pltpu.sync_copy(buf0, out_hbm.at[pl.ds(cbase, CHUNK)])

            @pl.when(cbase >= t)
            def _zero_chunk():
                pltpu.sync_copy(zeros_hbm.at[pl.ds(0, CHUNK)], buf0)
                pltpu.sync_copy(buf0, out_hbm.at[pl.ds(cbase, CHUNK)])

            @pl.when(jnp.logical_and(cbase < t, cbase + CHUNK > t))
            def _straddle_chunk():
                pltpu.sync_copy(pe_hbm.at[pl.ds(cbase, CHUNK)], buf0)

                def row_body(r, carry):
                    @pl.when(cbase + r >= t)
                    def _zero_row():
                        pltpu.sync_copy(zeros_hbm.at[0], buf0.at[r])

                    return carry

                lax.fori_loop(0, CHUNK, row_body, 0)
                pltpu.sync_copy(buf0, out_hbm.at[pl.ds(cbase, CHUNK)])


_sc_call = pl.kernel(
    _sc_body,
    mesh=_mesh,
    out_type=jax.ShapeDtypeStruct((MAX_LEN, DIM), jnp.float32),
    scratch_types=(
        [pltpu.VMEM((16,), jnp.int32)]
        + [pltpu.VMEM((CHUNK, DIM), jnp.float32)] * NBUF
        + [pltpu.SemaphoreType.DMA] * (2 * NBUF)
    ),
)


def kernel(pe, T):
    t_arr = jnp.full((16,), T, dtype=jnp.int32)
    zeros = jnp.zeros((CHUNK, DIM), dtype=jnp.float32)
    out = _sc_call(t_arr, pe, zeros)
    return out[None, :, :]


# speedup vs baseline: 1.0516x; 1.0516x over previous
import jax
import jax.numpy as jnp
from jax import lax
from jax.experimental import pallas as pl
from jax.experimental.pallas import tpu as pltpu
from jax.experimental.pallas import tpu_sc as plsc

MAX_LEN = 8192
DIM = 1024
NUM_WORKERS = 32
ROWS_PER_WORKER = MAX_LEN // NUM_WORKERS
CHUNK = 32
CHUNKS_PER_WORKER = ROWS_PER_WORKER // CHUNK
NBUF = 2

_mesh = plsc.VectorSubcoreMesh(core_axis_name="c", subcore_axis_name="s")


def _sc_body(t_hbm, pe_hbm, zeros_hbm, out_hbm,
             t_v, spmem, si0, si1, so0, so1):
    sid = lax.axis_index("s")
    wid = sid * 2 + lax.axis_index("c")
    base = wid * ROWS_PER_WORKER

    pltpu.sync_copy(t_hbm, t_v)
    t = t_v[...][0]

    bufs = tuple(spmem.at[sid, b] for b in range(NBUF))
    sin = (si0, si1)
    sout = (so0, so1)

    @pl.when(base + ROWS_PER_WORKER <= t)
    def _fast_copy():
        n = CHUNKS_PER_WORKER
        h_in = [None] * n
        h_out = [None] * n
        for i in range(n):
            b = i % NBUF
            if i >= NBUF:
                h_out[i - NBUF].wait()
            src = pe_hbm.at[pl.ds(base + i * CHUNK, CHUNK)]
            h_in[i] = pltpu.async_copy(src, bufs[b], sin[b])
            if i >= 1:
                h_in[i - 1].wait()
                pb = (i - 1) % NBUF
                dst = out_hbm.at[pl.ds(base + (i - 1) * CHUNK, CHUNK)]
                h_out[i - 1] = pltpu.async_copy(bufs[pb], dst, sout[pb])
        h_in[n - 1].wait()
        lb = (n - 1) % NBUF
        dst = out_hbm.at[pl.ds(base + (n - 1) * CHUNK, CHUNK)]
        h_out[n - 1] = pltpu.async_copy(bufs[lb], dst, sout[lb])
        for i in range(n - NBUF, n):
            h_out[i].wait()

    @pl.when(base + ROWS_PER_WORKER > t)
    def _masked_path():
        buf0 = bufs[0]
        for ci in range(CHUNKS_PER_WORKER):
            cbase = base + ci * CHUNK

            @pl.when(cbase + CHUNK <= t)
            def _copy_chunk():
                pltpu.sync_copy(pe_hbm.at[pl.ds(cbase, CHUNK)], buf0)
                pltpu.sync_copy(buf0, out_hbm.at[pl.ds(cbase, CHUNK)])

            @pl.when(cbase >= t)
            def _zero_chunk():
                pltpu.sync_copy(zeros_hbm.at[pl.ds(0, CHUNK)], buf0)
                pltpu.sync_copy(buf0, out_hbm.at[pl.ds(cbase, CHUNK)])

            @pl.when(jnp.logical_and(cbase < t, cbase + CHUNK > t))
            def _straddle_chunk():
                pltpu.sync_copy(pe_hbm.at[pl.ds(cbase, CHUNK)], buf0)

                def row_body(r, carry):
                    @pl.when(cbase + r >= t)
                    def _zero_row():
                        pltpu.sync_copy(zeros_hbm.at[0], buf0.at[r])

                    return carry

                lax.fori_loop(0, CHUNK, row_body, 0)
                pltpu.sync_copy(buf0, out_hbm.at[pl.ds(cbase, CHUNK)])


_sc_call = pl.kernel(
    _sc_body,
    mesh=_mesh,
    out_type=jax.ShapeDtypeStruct((MAX_LEN, DIM), jnp.float32),
    scratch_types=(
        [pltpu.VMEM((16,), jnp.int32),
         pltpu.VMEM_SHARED((16, NBUF, CHUNK, DIM), jnp.float32)]
        + [pltpu.SemaphoreType.DMA] * (2 * NBUF)
    ),
)


def kernel(pe, T):
    t_arr = jnp.full((16,), T, dtype=jnp.int32)
    zeros = jnp.zeros((CHUNK, DIM), dtype=jnp.float32)
    out = _sc_call(t_arr, pe, zeros)
    return out[None, :, :]
